# Initial kernel scaffold; baseline (speedup 1.0000x reference)
import functools

import jax
import jax.numpy as jnp
from jax.experimental import pallas as pl
from jax.experimental.pallas import tpu as pltpu

B, P, T = 16, 4000, 32
LINED = 72
BIG = 100000000.0


def _body(scal_ref, logits_ref, reg_ref, lines_ref, tlines_ref, tsc_ref,
          masksv_ref, matched_ref, assigned_ref,
          cost_sc, ious_sc, work_sc, elig_sc):
    wm1 = scal_ref[0, 0]
    wf = scal_ref[0, 1]
    kconst = scal_ref[0, 2]

    lg = logits_ref[0]
    x0 = lg[0:1, :]
    x1 = lg[1:2, :]
    m = jnp.maximum(x0, x1)
    e0 = jnp.exp(x0 - m)
    e1 = jnp.exp(x1 - m)
    p1 = e1 / (e0 + e1)
    c3 = 3.0 * (-jnp.log(jnp.maximum(p1, 1e-8)))

    p2 = reg_ref[0, 0:1, :]
    p3 = reg_ref[0, 1:2, :]
    p4 = reg_ref[0, 2:3, :]
    pls = lines_ref[0] * wm1

    num_gt = jnp.sum(masksv_ref[0])

    def g_body(g, _):
        t3 = tsc_ref[0, g, 0]
        t2 = tsc_ref[0, g, 1]
        t4 = tsc_ref[0, g, 2]
        mg = tsc_ref[0, g, 3]
        cx = jnp.abs(p3 - t3)
        cy = jnp.abs(p2 - t2)
        ct = jnp.abs(p4 - t4)
        r3 = 3.0 * ((cx + cy) + ct)

        gl = tlines_ref[0, g] * wm1
        inv = (gl < 0.0) | (gl >= wf)
        ad = jnp.abs(pls - gl)
        s = jnp.sum(jnp.where(inv, 0.0, ad), axis=0, keepdims=True)
        nv = jnp.sum(jnp.where(inv, 0.0, 1.0))
        a = 30.0 * nv
        li = (a - s) / ((a + s) + 1e-9)
        valid = mg > 0.0
        ious_sc[pl.ds(g, 1), :] = jnp.where(valid, li, 0.0)
        x = jnp.maximum(li, 1e-8)
        ic = jnp.where(x == 1e-8, kconst, -jnp.log(x))
        i3 = 3.0 * jnp.where(valid, ic, 0.0)
        tot = ((c3 + r3) + i3) + (100000.0 * jnp.where(valid, 0.0, 1.0))
        cost_sc[pl.ds(g, 1), :] = tot
        return 0

    jax.lax.fori_loop(0, T, g_body, 0, unroll=False)

    lane_i = jax.lax.broadcasted_iota(jnp.int32, (T, P), 1)
    sub_i = jax.lax.broadcasted_iota(jnp.int32, (T, P), 0)

    def a_body(s_, acc):
        iv = ious_sc[:, :]
        rm = jnp.max(iv, axis=1, keepdims=True)
        cand = jnp.where(iv == rm, lane_i, P)
        am = jnp.min(cand, axis=1, keepdims=True)
        sel = lane_i == am
        ious_sc[:, :] = jnp.where(sel, -jnp.inf, iv)
        return acc + rm

    acc = jax.lax.fori_loop(0, 10, a_body, jnp.zeros((T, 1), jnp.float32),
                            unroll=False)
    ks = jnp.clip(acc.astype(jnp.int32), 1, P)

    work_sc[:, :] = cost_sc[:, :]
    elig_sc[:, :] = jnp.zeros((T, P), jnp.int32)

    def b_body(s_, _):
        wv = work_sc[:, :]
        rm = jnp.min(wv, axis=1, keepdims=True)
        cand = jnp.where(wv == rm, lane_i, P)
        am = jnp.min(cand, axis=1, keepdims=True)
        sel = lane_i == am
        mark = sel & (s_ < ks)
        elig_sc[:, :] = jnp.where(mark, 1, elig_sc[:, :])
        work_sc[:, :] = jnp.where(sel, jnp.inf, wv)
        return 0

    jax.lax.fori_loop(0, 10, b_body, 0, unroll=False)

    gv = (jax.lax.broadcasted_iota(jnp.float32, (T, 1), 0) < num_gt)
    cond = (elig_sc[:, :] == 1) & gv
    cv = cost_sc[:, :]
    cm = jnp.where(cond, cv, BIG)
    mc = jnp.min(cm, axis=0, keepdims=True)
    cg = jnp.where(cond & (cv == mc), sub_i, T + 1)
    mg = jnp.min(cg, axis=0, keepdims=True)
    assigned = mc < BIG
    matched_ref[0] = jnp.where(assigned, mg, -1)
    assigned_ref[0] = assigned.astype(jnp.int32)


@jax.jit
def _run(preds, targets, masks, img_w_f, wm1_f, kconst):
    logits_t = jnp.transpose(preds[:, :, 0:2], (0, 2, 1))
    reg_t = jnp.transpose(preds[:, :, 2:5], (0, 2, 1))
    lines_t = jnp.transpose(preds[:, :, 6:], (0, 2, 1))
    tlines = targets[:, :, 6:].reshape(B, T, LINED, 1)
    tsc = jnp.stack(
        [targets[:, :, 3], targets[:, :, 2], targets[:, :, 4],
         masks.astype(jnp.float32)], axis=-1)
    masksv = masks.astype(jnp.float32).reshape(B, 1, T)
    scal = jnp.stack([wm1_f, img_w_f, kconst,
                      jnp.float32(0.0)]).reshape(1, 4)

    matched3, assigned3 = pl.pallas_call(
        _body,
        grid=(B,),
        in_specs=[
            pl.BlockSpec((1, 4), lambda b: (0, 0), memory_space=pltpu.SMEM),
            pl.BlockSpec((1, 2, P), lambda b: (b, 0, 0)),
            pl.BlockSpec((1, 3, P), lambda b: (b, 0, 0)),
            pl.BlockSpec((1, LINED, P), lambda b: (b, 0, 0)),
            pl.BlockSpec((1, T, LINED, 1), lambda b: (b, 0, 0, 0)),
            pl.BlockSpec((1, T, 4), lambda b: (b, 0, 0),
                         memory_space=pltpu.SMEM),
            pl.BlockSpec((1, 1, T), lambda b: (b, 0, 0)),
        ],
        out_specs=[
            pl.BlockSpec((1, 1, P), lambda b: (b, 0, 0)),
            pl.BlockSpec((1, 1, P), lambda b: (b, 0, 0)),
        ],
        out_shape=[
            jax.ShapeDtypeStruct((B, 1, P), jnp.int32),
            jax.ShapeDtypeStruct((B, 1, P), jnp.int32),
        ],
        scratch_shapes=[
            pltpu.VMEM((T, P), jnp.float32),
            pltpu.VMEM((T, P), jnp.float32),
            pltpu.VMEM((T, P), jnp.float32),
            pltpu.VMEM((T, P), jnp.int32),
        ],
    )(scal, logits_t, reg_t, lines_t, tlines, tsc, masksv)

    matched = matched3.reshape(B, P)
    assigned = assigned3.reshape(B, P) != 0
    return assigned, matched


def kernel(preds, targets, masks, img_w, img_h):
    img_w_f = jnp.asarray(img_w).astype(jnp.float32)
    wm1_f = (jnp.asarray(img_w) - 1).astype(jnp.float32)
    kconst = -jnp.log(jnp.clip(jnp.float32(1e-8), 1e-08, None))
    return _run(preds, targets, masks, img_w_f, wm1_f, kconst)


# speedup vs baseline: 8.8524x; 8.8524x over previous
import functools

import jax
import jax.numpy as jnp
from jax.experimental import pallas as pl
from jax.experimental.pallas import tpu as pltpu

B, P, T = 16, 4000, 32
LINED = 72
BIG = 100000000.0


def _body(scal_ref, logits_ref, reg_ref, lines_ref, tlines_ref, tsc_ref,
          masksv_ref, matched_ref, assigned_ref,
          cost_sc, ious_sc, work_sc, elig_sc):
    wm1 = scal_ref[0, 0]
    wf = scal_ref[0, 1]
    kconst = scal_ref[0, 2]

    lg = logits_ref[0]
    x0 = lg[0:1, :]
    x1 = lg[1:2, :]
    m = jnp.maximum(x0, x1)
    e0 = jnp.exp(x0 - m)
    e1 = jnp.exp(x1 - m)
    p1 = e1 / (e0 + e1)
    c3 = 3.0 * (-jnp.log(jnp.maximum(p1, 1e-8)))

    p2 = reg_ref[0, 0:1, :]
    p3 = reg_ref[0, 1:2, :]
    p4 = reg_ref[0, 2:3, :]
    pls = lines_ref[0] * wm1

    num_gt = jnp.sum(masksv_ref[0])

    def g_body(g, _):
        t3 = tsc_ref[0, g, 0]
        t2 = tsc_ref[0, g, 1]
        t4 = tsc_ref[0, g, 2]
        mg = tsc_ref[0, g, 3]
        cx = jnp.abs(p3 - t3)
        cy = jnp.abs(p2 - t2)
        ct = jnp.abs(p4 - t4)
        r3 = 3.0 * ((cx + cy) + ct)

        gl = tlines_ref[0, g] * wm1
        inv = (gl < 0.0) | (gl >= wf)
        ad = jnp.abs(pls - gl)
        s = jnp.sum(jnp.where(inv, 0.0, ad), axis=0, keepdims=True)
        nv = jnp.sum(jnp.where(inv, 0.0, 1.0))
        a = 30.0 * nv
        li = (a - s) / ((a + s) + 1e-9)
        valid = mg > 0.0
        ious_sc[pl.ds(g, 1), :] = jnp.where(valid, li, 0.0)
        x = jnp.maximum(li, 1e-8)
        ic = jnp.where(x == 1e-8, kconst, -jnp.log(x))
        i3 = 3.0 * jnp.where(valid, ic, 0.0)
        tot = ((c3 + r3) + i3) + (100000.0 * jnp.where(valid, 0.0, 1.0))
        cost_sc[pl.ds(g, 1), :] = tot
        return 0

    jax.lax.fori_loop(0, T, g_body, 0, unroll=False)

    lane_i = jax.lax.broadcasted_iota(jnp.int32, (T, P), 1)
    sub_i = jax.lax.broadcasted_iota(jnp.int32, (T, P), 0)

    def a_body(s_, acc):
        iv = ious_sc[:, :]
        rm = jnp.max(iv, axis=1, keepdims=True)
        cand = jnp.where(iv == rm, lane_i, P)
        am = jnp.min(cand, axis=1, keepdims=True)
        sel = lane_i == am
        ious_sc[:, :] = jnp.where(sel, -jnp.inf, iv)
        return acc + rm

    acc = jax.lax.fori_loop(0, 10, a_body, jnp.zeros((T, 1), jnp.float32),
                            unroll=False)
    ks = jnp.clip(acc.astype(jnp.int32), 1, P)

    work_sc[:, :] = cost_sc[:, :]
    elig_sc[:, :] = jnp.zeros((T, P), jnp.int32)

    def b_body(s_, _):
        wv = work_sc[:, :]
        rm = jnp.min(wv, axis=1, keepdims=True)
        cand = jnp.where(wv == rm, lane_i, P)
        am = jnp.min(cand, axis=1, keepdims=True)
        sel = lane_i == am
        mark = sel & (s_ < ks)
        elig_sc[:, :] = jnp.where(mark, 1, elig_sc[:, :])
        work_sc[:, :] = jnp.where(sel, jnp.inf, wv)
        return 0

    jax.lax.fori_loop(0, 10, b_body, 0, unroll=False)

    gv = (jax.lax.broadcasted_iota(jnp.int32, (T, 1), 0).astype(jnp.float32)
          < num_gt)
    cond = (elig_sc[:, :] == 1) & gv
    cv = cost_sc[:, :]
    cm = jnp.where(cond, cv, BIG)
    mc = jnp.min(cm, axis=0, keepdims=True)
    cg = jnp.where(cond & (cv == mc), sub_i, T + 1)
    mg = jnp.min(cg, axis=0, keepdims=True)
    assigned = mc < BIG
    matched_ref[0] = jnp.where(assigned, mg, -1)
    assigned_ref[0] = assigned.astype(jnp.int32)


@jax.jit
def _run(preds, targets, masks, img_w_f, wm1_f, kconst):
    logits_t = jnp.transpose(preds[:, :, 0:2], (0, 2, 1))
    reg_t = jnp.transpose(preds[:, :, 2:5], (0, 2, 1))
    lines_t = jnp.transpose(preds[:, :, 6:], (0, 2, 1))
    tlines = targets[:, :, 6:].reshape(B, T, LINED, 1)
    tsc = jnp.stack(
        [targets[:, :, 3], targets[:, :, 2], targets[:, :, 4],
         masks.astype(jnp.float32)], axis=-1)
    masksv = masks.astype(jnp.float32).reshape(B, 1, T)
    scal = jnp.stack([wm1_f, img_w_f, kconst,
                      jnp.float32(0.0)]).reshape(1, 4)

    matched3, assigned3 = pl.pallas_call(
        _body,
        grid=(B,),
        in_specs=[
            pl.BlockSpec((1, 4), lambda b: (0, 0), memory_space=pltpu.SMEM),
            pl.BlockSpec((1, 2, P), lambda b: (b, 0, 0)),
            pl.BlockSpec((1, 3, P), lambda b: (b, 0, 0)),
            pl.BlockSpec((1, LINED, P), lambda b: (b, 0, 0)),
            pl.BlockSpec((1, T, LINED, 1), lambda b: (b, 0, 0, 0)),
            pl.BlockSpec((1, T, 4), lambda b: (b, 0, 0),
                         memory_space=pltpu.SMEM),
            pl.BlockSpec((1, 1, T), lambda b: (b, 0, 0)),
        ],
        out_specs=[
            pl.BlockSpec((1, 1, P), lambda b: (b, 0, 0)),
            pl.BlockSpec((1, 1, P), lambda b: (b, 0, 0)),
        ],
        out_shape=[
            jax.ShapeDtypeStruct((B, 1, P), jnp.int32),
            jax.ShapeDtypeStruct((B, 1, P), jnp.int32),
        ],
        scratch_shapes=[
            pltpu.VMEM((T, P), jnp.float32),
            pltpu.VMEM((T, P), jnp.float32),
            pltpu.VMEM((T, P), jnp.float32),
            pltpu.VMEM((T, P), jnp.int32),
        ],
    )(scal, logits_t, reg_t, lines_t, tlines, tsc, masksv)

    matched = matched3.reshape(B, P)
    assigned = assigned3.reshape(B, P) != 0
    return assigned, matched


def kernel(preds, targets, masks, img_w, img_h):
    img_w_f = jnp.asarray(img_w).astype(jnp.float32)
    wm1_f = (jnp.asarray(img_w) - 1).astype(jnp.float32)
    kconst = -jnp.log(jnp.clip(jnp.float32(1e-8), 1e-08, None))
    return _run(preds, targets, masks, img_w_f, wm1_f, kconst)


# speedup vs baseline: 12.7210x; 1.4370x over previous
import functools

import jax
import jax.numpy as jnp
from jax.experimental import pallas as pl
from jax.experimental.pallas import tpu as pltpu

B, P, T = 16, 4000, 32
LINED = 72
BIG = 100000000.0


def _body(scal_ref, logits_ref, reg_ref, lines_ref, tlines_ref, tsc_ref,
          masksv_ref, matched_ref, assigned_ref,
          cost_sc, ious_sc, elig_sc):
    wm1 = scal_ref[0, 0]
    wf = scal_ref[0, 1]
    kconst = scal_ref[0, 2]

    lg = logits_ref[0]
    x0 = lg[0:1, :]
    x1 = lg[1:2, :]
    m = jnp.maximum(x0, x1)
    e0 = jnp.exp(x0 - m)
    e1 = jnp.exp(x1 - m)
    p1 = e1 / (e0 + e1)
    c3 = 3.0 * (-jnp.log(jnp.maximum(p1, 1e-8)))

    p2 = reg_ref[0, 0:1, :]
    p3 = reg_ref[0, 1:2, :]
    p4 = reg_ref[0, 2:3, :]
    pls = lines_ref[0] * wm1

    num_gt = jnp.sum(masksv_ref[0])

    def g_body(g, _):
        t3 = tsc_ref[0, g, 0]
        t2 = tsc_ref[0, g, 1]
        t4 = tsc_ref[0, g, 2]
        mg = tsc_ref[0, g, 3]
        cx = jnp.abs(p3 - t3)
        cy = jnp.abs(p2 - t2)
        ct = jnp.abs(p4 - t4)
        r3 = 3.0 * ((cx + cy) + ct)

        gl = tlines_ref[0, g] * wm1
        inv = (gl < 0.0) | (gl >= wf)
        ad = jnp.abs(pls - gl)
        s = jnp.sum(jnp.where(inv, 0.0, ad), axis=0, keepdims=True)
        nv = jnp.sum(jnp.where(inv, 0.0, 1.0))
        a = 30.0 * nv
        li = (a - s) / ((a + s) + 1e-9)
        valid = mg > 0.0
        ious_sc[pl.ds(g, 1), :] = jnp.where(valid, li, 0.0)
        x = jnp.maximum(li, 1e-8)
        ic = jnp.where(x == 1e-8, kconst, -jnp.log(x))
        i3 = 3.0 * jnp.where(valid, ic, 0.0)
        tot = ((c3 + r3) + i3) + (100000.0 * jnp.where(valid, 0.0, 1.0))
        cost_sc[pl.ds(g, 1), :] = tot
        return 0

    jax.lax.fori_loop(0, T, g_body, 0, unroll=False)

    lane_f = jax.lax.broadcasted_iota(jnp.int32, (T, P), 1).astype(jnp.float32)
    sub_i = jax.lax.broadcasted_iota(jnp.int32, (T, P), 0)

    gm = jnp.max(ious_sc[:, :])

    def a_body(s_, acc):
        iv = ious_sc[:, :]
        rm = jnp.max(iv, axis=1, keepdims=True)
        cand = jnp.where(iv == rm, lane_f, 1.0e9)
        am = jnp.min(cand, axis=1, keepdims=True)
        sel = lane_f == am
        ious_sc[:, :] = jnp.where(sel, -jnp.inf, iv)
        return acc + rm

    asteps = jnp.where(gm < 0.2, 0, 10)
    acc = jax.lax.fori_loop(0, asteps, a_body,
                            jnp.zeros((T, 1), jnp.float32))
    ks = jnp.clip(acc.astype(jnp.int32), 1, P)
    kmax = jnp.max(ks)

    elig_sc[:, :] = jnp.full((T, P), 127, jnp.int32)

    def b_body(s_, _):
        ss = elig_sc[:, :]
        cvv = cost_sc[:, :]
        wv = jnp.where(ss < 127, jnp.inf, cvv)
        rm = jnp.min(wv, axis=1, keepdims=True)
        cand = jnp.where(wv == rm, lane_f, 1.0e9)
        am = jnp.min(cand, axis=1, keepdims=True)
        sel = lane_f == am
        elig_sc[:, :] = jnp.where(sel, s_, ss)
        return 0

    jax.lax.fori_loop(0, kmax, b_body, 0)

    gv = (jax.lax.broadcasted_iota(jnp.int32, (T, 1), 0).astype(jnp.float32)
          < num_gt)
    cond = (elig_sc[:, :] < ks) & gv
    cv = cost_sc[:, :]
    cm = jnp.where(cond, cv, BIG)
    mc = jnp.min(cm, axis=0, keepdims=True)
    cg = jnp.where(cond & (cv == mc), sub_i, T + 1)
    mg = jnp.min(cg, axis=0, keepdims=True)
    assigned = mc < BIG
    matched_ref[0] = jnp.where(assigned, mg, -1)
    assigned_ref[0] = assigned.astype(jnp.int32)


@jax.jit
def _run(preds, targets, masks, img_w_f, wm1_f, kconst):
    logits_t = jnp.transpose(preds[:, :, 0:2], (0, 2, 1))
    reg_t = jnp.transpose(preds[:, :, 2:5], (0, 2, 1))
    lines_t = jnp.transpose(preds[:, :, 6:], (0, 2, 1))
    tlines = targets[:, :, 6:].reshape(B, T, LINED, 1)
    tsc = jnp.stack(
        [targets[:, :, 3], targets[:, :, 2], targets[:, :, 4],
         masks.astype(jnp.float32)], axis=-1)
    masksv = masks.astype(jnp.float32).reshape(B, 1, T)
    scal = jnp.stack([wm1_f, img_w_f, kconst,
                      jnp.float32(0.0)]).reshape(1, 4)

    matched3, assigned3 = pl.pallas_call(
        _body,
        grid=(B,),
        in_specs=[
            pl.BlockSpec((1, 4), lambda b: (0, 0), memory_space=pltpu.SMEM),
            pl.BlockSpec((1, 2, P), lambda b: (b, 0, 0)),
            pl.BlockSpec((1, 3, P), lambda b: (b, 0, 0)),
            pl.BlockSpec((1, LINED, P), lambda b: (b, 0, 0)),
            pl.BlockSpec((1, T, LINED, 1), lambda b: (b, 0, 0, 0)),
            pl.BlockSpec((1, T, 4), lambda b: (b, 0, 0),
                         memory_space=pltpu.SMEM),
            pl.BlockSpec((1, 1, T), lambda b: (b, 0, 0)),
        ],
        out_specs=[
            pl.BlockSpec((1, 1, P), lambda b: (b, 0, 0)),
            pl.BlockSpec((1, 1, P), lambda b: (b, 0, 0)),
        ],
        out_shape=[
            jax.ShapeDtypeStruct((B, 1, P), jnp.int32),
            jax.ShapeDtypeStruct((B, 1, P), jnp.int32),
        ],
        scratch_shapes=[
            pltpu.VMEM((T, P), jnp.float32),
            pltpu.VMEM((T, P), jnp.float32),
            pltpu.VMEM((T, P), jnp.int32),
        ],
    )(scal, logits_t, reg_t, lines_t, tlines, tsc, masksv)

    matched = matched3.reshape(B, P)
    assigned = assigned3.reshape(B, P) != 0
    return assigned, matched


def kernel(preds, targets, masks, img_w, img_h):
    img_w_f = jnp.asarray(img_w).astype(jnp.float32)
    wm1_f = (jnp.asarray(img_w) - 1).astype(jnp.float32)
    kconst = -jnp.log(jnp.clip(jnp.float32(1e-8), 1e-08, None))
    return _run(preds, targets, masks, img_w_f, wm1_f, kconst)


# speedup vs baseline: 26.5395x; 2.0863x over previous
import functools

import jax
import jax.numpy as jnp
from jax.experimental import pallas as pl
from jax.experimental.pallas import tpu as pltpu

B, P, T = 16, 4000, 32
LINED = 72
BIG = 100000000.0


def _body(scal_ref, logits_ref, reg_ref, lines_ref, tlines_ref, tscv_ref,
          masksv_ref, matched_ref, assigned_ref,
          cost_sc, ious_sc, elig_sc):
    wm1 = scal_ref[0, 0]
    wf = scal_ref[0, 1]
    kconst = scal_ref[0, 2]

    lg = logits_ref[0]
    x0 = lg[0:1, :]
    x1 = lg[1:2, :]
    m = jnp.maximum(x0, x1)
    e0 = jnp.exp(x0 - m)
    e1 = jnp.exp(x1 - m)
    p1 = e1 / (e0 + e1)
    c3 = 3.0 * (-jnp.log(jnp.maximum(p1, 1e-8)))

    p2 = reg_ref[0, 0:1, :]
    p3 = reg_ref[0, 1:2, :]
    p4 = reg_ref[0, 2:3, :]
    p2b = jnp.broadcast_to(p2, (8, P))
    p3b = jnp.broadcast_to(p3, (8, P))
    p4b = jnp.broadcast_to(p4, (8, P))
    c3b = jnp.broadcast_to(c3, (8, P))
    lb = lines_ref[0]

    num_gt = jnp.sum(masksv_ref[0])

    def g_body(gi, _):
        g8 = gi * 8
        s_rows = []
        nv_rows = []
        for j in range(8):
            gl = tlines_ref[0, g8 + j]
            gls = gl * wm1
            inv = (gls < 0.0) | (gls >= wf)
            v = jnp.where(inv, 0.0, 1.0)
            ad = jnp.abs(lb - gl.astype(jnp.bfloat16))
            sj = jax.lax.dot_general(
                v.astype(jnp.bfloat16), ad,
                (((0,), (0,)), ((), ())),
                preferred_element_type=jnp.float32)
            s_rows.append(sj)
            nv_rows.append(jnp.sum(v).reshape(1, 1))
        s8 = wm1 * jnp.concatenate(s_rows, axis=0)
        nv8 = jnp.concatenate(nv_rows, axis=0)
        a8 = 30.0 * nv8
        li8 = (a8 - s8) / ((a8 + s8) + 1e-9)
        m8 = tscv_ref[0, pl.ds(g8, 8), 3:4]
        valid8 = m8 > 0.0
        ious_sc[pl.ds(g8, 8), :] = jnp.where(valid8, li8, 0.0)
        x8 = jnp.maximum(li8, 1e-8)
        ic8 = jnp.where(x8 == 1e-8, kconst, -jnp.log(x8))
        i38 = 3.0 * jnp.where(valid8, ic8, 0.0)
        t3c = tscv_ref[0, pl.ds(g8, 8), 0:1]
        t2c = tscv_ref[0, pl.ds(g8, 8), 1:2]
        t4c = tscv_ref[0, pl.ds(g8, 8), 2:3]
        cx = jnp.abs(p3b - t3c)
        cy = jnp.abs(p2b - t2c)
        ct = jnp.abs(p4b - t4c)
        r38 = 3.0 * ((cx + cy) + ct)
        tot8 = ((c3b + r38) + i38) + (100000.0 * jnp.where(valid8, 0.0, 1.0))
        cost_sc[pl.ds(g8, 8), :] = tot8
        return 0

    jax.lax.fori_loop(0, T // 8, g_body, 0, unroll=False)

    lane_f = jax.lax.broadcasted_iota(jnp.int32, (T, P), 1).astype(jnp.float32)
    sub_i = jax.lax.broadcasted_iota(jnp.int32, (T, P), 0)

    gm = jnp.max(ious_sc[:, :])

    def a_body(s_, acc):
        iv = ious_sc[:, :]
        rm = jnp.max(iv, axis=1, keepdims=True)
        cand = jnp.where(iv == rm, lane_f, 1.0e9)
        am = jnp.min(cand, axis=1, keepdims=True)
        sel = lane_f == am
        ious_sc[:, :] = jnp.where(sel, -jnp.inf, iv)
        return acc + rm

    asteps = jnp.where(gm < 0.2, 0, 10)
    acc = jax.lax.fori_loop(0, asteps, a_body,
                            jnp.zeros((T, 1), jnp.float32))
    ks = jnp.clip(acc.astype(jnp.int32), 1, P)
    kmax = jnp.max(ks)

    elig_sc[:, :] = jnp.full((T, P), 127, jnp.int32)

    def b_body(s_, _):
        ss = elig_sc[:, :]
        cvv = cost_sc[:, :]
        wv = jnp.where(ss < 127, jnp.inf, cvv)
        rm = jnp.min(wv, axis=1, keepdims=True)
        cand = jnp.where(wv == rm, lane_f, 1.0e9)
        am = jnp.min(cand, axis=1, keepdims=True)
        sel = lane_f == am
        elig_sc[:, :] = jnp.where(sel, s_, ss)
        return 0

    jax.lax.fori_loop(0, kmax, b_body, 0)

    gv = (jax.lax.broadcasted_iota(jnp.int32, (T, 1), 0).astype(jnp.float32)
          < num_gt)
    cond = (elig_sc[:, :] < ks) & gv
    cv = cost_sc[:, :]
    cm = jnp.where(cond, cv, BIG)
    mc = jnp.min(cm, axis=0, keepdims=True)
    cg = jnp.where(cond & (cv == mc), sub_i, T + 1)
    mg = jnp.min(cg, axis=0, keepdims=True)
    assigned = mc < BIG
    matched_ref[0] = jnp.where(assigned, mg, -1)
    assigned_ref[0] = assigned.astype(jnp.int32)


@jax.jit
def _run(preds, targets, masks, img_w_f, wm1_f, kconst):
    logits_t = jnp.transpose(preds[:, :, 0:2], (0, 2, 1))
    reg_t = jnp.transpose(preds[:, :, 2:5], (0, 2, 1))
    lines_bf = jnp.transpose(preds[:, :, 6:], (0, 2, 1)).astype(jnp.bfloat16)
    tlines = targets[:, :, 6:].reshape(B, T, LINED, 1)
    tsc = jnp.stack(
        [targets[:, :, 3], targets[:, :, 2], targets[:, :, 4],
         masks.astype(jnp.float32)], axis=-1)
    masksv = masks.astype(jnp.float32).reshape(B, 1, T)
    scal = jnp.stack([wm1_f, img_w_f, kconst,
                      jnp.float32(0.0)]).reshape(1, 4)

    matched3, assigned3 = pl.pallas_call(
        _body,
        grid=(B,),
        in_specs=[
            pl.BlockSpec((1, 4), lambda b: (0, 0), memory_space=pltpu.SMEM),
            pl.BlockSpec((1, 2, P), lambda b: (b, 0, 0)),
            pl.BlockSpec((1, 3, P), lambda b: (b, 0, 0)),
            pl.BlockSpec((1, LINED, P), lambda b: (b, 0, 0)),
            pl.BlockSpec((1, T, LINED, 1), lambda b: (b, 0, 0, 0)),
            pl.BlockSpec((1, T, 4), lambda b: (b, 0, 0)),
            pl.BlockSpec((1, 1, T), lambda b: (b, 0, 0)),
        ],
        out_specs=[
            pl.BlockSpec((1, 1, P), lambda b: (b, 0, 0)),
            pl.BlockSpec((1, 1, P), lambda b: (b, 0, 0)),
        ],
        out_shape=[
            jax.ShapeDtypeStruct((B, 1, P), jnp.int32),
            jax.ShapeDtypeStruct((B, 1, P), jnp.int32),
        ],
        scratch_shapes=[
            pltpu.VMEM((T, P), jnp.float32),
            pltpu.VMEM((T, P), jnp.float32),
            pltpu.VMEM((T, P), jnp.int32),
        ],
    )(scal, logits_t, reg_t, lines_bf, tlines, tsc, masksv)

    matched = matched3.reshape(B, P)
    assigned = assigned3.reshape(B, P) != 0
    return assigned, matched


def kernel(preds, targets, masks, img_w, img_h):
    img_w_f = jnp.asarray(img_w).astype(jnp.float32)
    wm1_f = (jnp.asarray(img_w) - 1).astype(jnp.float32)
    kconst = -jnp.log(jnp.clip(jnp.float32(1e-8), 1e-08, None))
    return _run(preds, targets, masks, img_w_f, wm1_f, kconst)
